# baseline (speedup 1.0000x reference)
import functools

import jax
import jax.numpy as jnp
from jax import lax
from jax.experimental import pallas as pl
from jax.experimental.pallas import tpu as pltpu
from jax.experimental.pallas import tpu_sc as plsc

D = 32
W = 128
B = 16384
NC, NS = 2, 16
NW = NC * NS
BPW = B // NW
RPB = 128
NCH = BPW // RPB
L = 16
NG = RPB // L
MARGIN = 1.0


def _sc_body(ent_hbm, rel_hbm, time_hbm, idx_hbm, sub_hbm,
             scores_out, loss_out,
             idx_v, sub_v, w_rows, h_rows, r_rows, t_rows, nh_rows, nt_rows,
             st, ut, wt, pos_buf, neg_buf, loss_buf, sem):
    wid = lax.axis_index("s") * NC + lax.axis_index("c")
    base = wid * BPW

    pltpu.sync_copy(idx_hbm.at[:, pl.ds(base, BPW)], idx_v)
    pltpu.sync_copy(sub_hbm.at[:, pl.ds(base, BPW)], sub_v)

    tables = (time_hbm, ent_hbm, rel_hbm, ent_hbm, ent_hbm, ent_hbm)
    dests = (w_rows, h_rows, r_rows, t_rows, nh_rows, nt_rows)
    zeros = jnp.zeros((L,), jnp.float32)
    lanes = lax.iota(jnp.int32, L)

    def chunk_body(c, lacc_c):
        cbase = c * RPB
        copies = []
        for a in range(6):
            copies.append(pltpu.async_copy(
                tables[a].at[idx_v.at[a, pl.ds(cbase, RPB)]],
                dests[a], sem))
        for cp in copies:
            cp.wait()

        def group_body(g, lacc):
            ridx = g * L + lanes
            gbase = cbase + g * L
            sh = sub_v[0, pl.ds(gbase, L)]
            stt = sub_v[1, pl.ds(gbase, L)]
            snh = sub_v[2, pl.ds(gbase, L)]
            snt = sub_v[3, pl.ds(gbase, L)]
            q = zeros
            dsum = zeros
            dusum = zeros
            for d in range(D):
                cidx = jnp.full((L,), d, jnp.int32)
                w_d = plsc.load_gather(w_rows, [ridx, cidx])
                r_d = plsc.load_gather(r_rows, [ridx, cidx])
                h_d = plsc.load_gather(h_rows, [ridx, sh + d])
                t_d = plsc.load_gather(t_rows, [ridx, stt + d])
                x_d = plsc.load_gather(nh_rows, [ridx, snh + d])
                y_d = plsc.load_gather(nt_rows, [ridx, snt + d])
                s_d = h_d + r_d - t_d
                u_d = x_d + r_d - y_d
                q = q + w_d * w_d
                dsum = dsum + w_d * s_d
                dusum = dusum + w_d * u_d
                wt[pl.ds(d * L, L)] = w_d
                st[pl.ds(d * L, L)] = s_d
                ut[pl.ds(d * L, L)] = u_d
            qi = 1.0 / (q + 1e-30)
            cs = dsum * qi
            cu = dusum * qi
            pos = zeros
            neg = zeros
            for d in range(D):
                w_d = wt[pl.ds(d * L, L)]
                s_d = st[pl.ds(d * L, L)]
                u_d = ut[pl.ds(d * L, L)]
                pos = pos + jnp.abs(s_d - cs * w_d)
                neg = neg + jnp.abs(u_d - cu * w_d)
            pos_buf[pl.ds(gbase, L)] = pos
            neg_buf[pl.ds(gbase, L)] = neg
            return lacc + jnp.maximum(pos + MARGIN - neg, 0.0)

        return lax.fori_loop(0, NG, group_body, lacc_c)

    lacc = lax.fori_loop(0, NCH, chunk_body, zeros)

    pltpu.sync_copy(pos_buf, scores_out.at[0, pl.ds(base, BPW)])
    pltpu.sync_copy(neg_buf, scores_out.at[1, pl.ds(base, BPW)])
    loss_buf[...] = lacc
    pltpu.sync_copy(loss_buf, loss_out.at[pl.ds(wid * L, L)])


_sc_call = functools.partial(
    pl.kernel,
    out_type=(
        jax.ShapeDtypeStruct((2, B), jnp.float32),
        jax.ShapeDtypeStruct((NW * L,), jnp.float32),
    ),
    mesh=plsc.VectorSubcoreMesh(core_axis_name="c", subcore_axis_name="s"),
    compiler_params=pltpu.CompilerParams(
        needs_layout_passes=False, use_tc_tiling_on_sc=False,
        disable_bounds_checks=True),
    scratch_types=[
        pltpu.VMEM((6, BPW), jnp.int32),
        pltpu.VMEM((4, BPW), jnp.int32),
        pltpu.VMEM((RPB, W), jnp.float32),
        pltpu.VMEM((RPB, W), jnp.float32),
        pltpu.VMEM((RPB, W), jnp.float32),
        pltpu.VMEM((RPB, W), jnp.float32),
        pltpu.VMEM((RPB, W), jnp.float32),
        pltpu.VMEM((RPB, W), jnp.float32),
        pltpu.VMEM((D * L,), jnp.float32),
        pltpu.VMEM((D * L,), jnp.float32),
        pltpu.VMEM((D * L,), jnp.float32),
        pltpu.VMEM((BPW,), jnp.float32),
        pltpu.VMEM((BPW,), jnp.float32),
        pltpu.VMEM((L,), jnp.float32),
        pltpu.SemaphoreType.DMA,
    ],
)(_sc_body)


CB = 8192


def _tc_repack_body(in_ref, out_ref):
    y = jnp.transpose(in_ref[...])
    y4 = jnp.reshape(y, (CB // 4, 4, D))
    out_ref[...] = jnp.concatenate([y4[:, k, :] for k in range(4)], axis=1)


def _tc_repack(ent_cm, n_ent):
    grid = (n_ent + CB - 1) // CB
    return pl.pallas_call(
        _tc_repack_body,
        grid=(grid,),
        in_specs=[pl.BlockSpec((D, CB), lambda j: (0, j))],
        out_specs=pl.BlockSpec((CB // 4, W), lambda j: (j, 0)),
        out_shape=jax.ShapeDtypeStruct((n_ent // 4, W), jnp.float32),
    )(ent_cm)


def kernel(ent_embed, rel_embed, time_embed, pos_h, pos_r, pos_t,
           neg_h, neg_t, time_idx):
    ent_r = _tc_repack(ent_embed.T, ent_embed.shape[0])
    rel_p = jnp.pad(rel_embed, ((0, 0), (0, W - D)))
    time_p = jnp.pad(time_embed, ((0, 0), (0, W - D)))
    idx_all = jnp.stack([time_idx, pos_h >> 2, pos_r, pos_t >> 2,
                         neg_h >> 2, neg_t >> 2], axis=0)
    sub_all = jnp.stack([(pos_h & 3) * D, (pos_t & 3) * D,
                         (neg_h & 3) * D, (neg_t & 3) * D], axis=0)
    scores, loss_parts = _sc_call(ent_r, rel_p, time_p, idx_all, sub_all)
    return scores, jnp.sum(loss_parts)


# speedup vs baseline: 1.3858x; 1.3858x over previous
import functools

import jax
import jax.numpy as jnp
from jax import lax
from jax.experimental import pallas as pl
from jax.experimental.pallas import tpu as pltpu
from jax.experimental.pallas import tpu_sc as plsc

D = 32
W = 128
B = 16384
NC, NS = 2, 16
NW = NC * NS
BPW = B // NW
RPB = 128
NCH = BPW // RPB
L = 16
NG = RPB // L
MARGIN = 1.0


def _sc_body(ent_hbm, rel_hbm, time_hbm, idx_hbm, sub_hbm,
             scores_out, loss_out,
             idx_v, sub_v, w_rows, h_rows, r_rows, t_rows, nh_rows, nt_rows,
             st, ut, wt, pos_buf, neg_buf, loss_buf, sem):
    wid = lax.axis_index("s") * NC + lax.axis_index("c")
    base = wid * BPW

    pltpu.sync_copy(idx_hbm.at[:, pl.ds(base, BPW)], idx_v)
    pltpu.sync_copy(sub_hbm.at[:, pl.ds(base, BPW)], sub_v)

    tables = (time_hbm, ent_hbm, rel_hbm, ent_hbm, ent_hbm, ent_hbm)
    dests = (w_rows, h_rows, r_rows, t_rows, nh_rows, nt_rows)
    zeros = jnp.zeros((L,), jnp.float32)
    lanes = lax.iota(jnp.int32, L)

    def chunk_body(c, lacc_c):
        cbase = c * RPB
        copies = []
        for a in range(6):
            copies.append(pltpu.async_copy(
                tables[a].at[idx_v.at[a, pl.ds(cbase, RPB)]],
                dests[a], sem))
        for cp in copies:
            cp.wait()

        def group_body(g, lacc):
            ridx = g * L + lanes
            gbase = cbase + g * L
            sh = sub_v[0, pl.ds(gbase, L)]
            stt = sub_v[1, pl.ds(gbase, L)]
            snh = sub_v[2, pl.ds(gbase, L)]
            snt = sub_v[3, pl.ds(gbase, L)]
            q = zeros
            dsum = zeros
            dusum = zeros
            for d in range(D):
                cidx = jnp.full((L,), d, jnp.int32)
                w_d = plsc.load_gather(w_rows, [ridx, cidx])
                r_d = plsc.load_gather(r_rows, [ridx, cidx])
                h_d = plsc.load_gather(h_rows, [ridx, sh + d])
                t_d = plsc.load_gather(t_rows, [ridx, stt + d])
                x_d = plsc.load_gather(nh_rows, [ridx, snh + d])
                y_d = plsc.load_gather(nt_rows, [ridx, snt + d])
                s_d = h_d + r_d - t_d
                u_d = x_d + r_d - y_d
                q = q + w_d * w_d
                dsum = dsum + w_d * s_d
                dusum = dusum + w_d * u_d
                wt[pl.ds(d * L, L)] = w_d
                st[pl.ds(d * L, L)] = s_d
                ut[pl.ds(d * L, L)] = u_d
            qi = 1.0 / (q + 1e-30)
            cs = dsum * qi
            cu = dusum * qi
            pos = zeros
            neg = zeros
            for d in range(D):
                w_d = wt[pl.ds(d * L, L)]
                s_d = st[pl.ds(d * L, L)]
                u_d = ut[pl.ds(d * L, L)]
                pos = pos + jnp.abs(s_d - cs * w_d)
                neg = neg + jnp.abs(u_d - cu * w_d)
            pos_buf[pl.ds(gbase, L)] = pos
            neg_buf[pl.ds(gbase, L)] = neg
            return lacc + jnp.maximum(pos + MARGIN - neg, 0.0)

        return lax.fori_loop(0, NG, group_body, lacc_c)

    lacc = lax.fori_loop(0, NCH, chunk_body, zeros)

    pltpu.sync_copy(pos_buf, scores_out.at[0, pl.ds(base, BPW)])
    pltpu.sync_copy(neg_buf, scores_out.at[1, pl.ds(base, BPW)])
    loss_buf[...] = lacc
    pltpu.sync_copy(loss_buf, loss_out.at[pl.ds(wid * L, L)])


_sc_call = functools.partial(
    pl.kernel,
    out_type=(
        jax.ShapeDtypeStruct((2, B), jnp.float32),
        jax.ShapeDtypeStruct((NW * L,), jnp.float32),
    ),
    mesh=plsc.VectorSubcoreMesh(core_axis_name="c", subcore_axis_name="s"),
    compiler_params=pltpu.CompilerParams(
        needs_layout_passes=False, use_tc_tiling_on_sc=False,
        disable_bounds_checks=True),
    scratch_types=[
        pltpu.VMEM((6, BPW), jnp.int32),
        pltpu.VMEM((4, BPW), jnp.int32),
        pltpu.VMEM((RPB, W), jnp.float32),
        pltpu.VMEM((RPB, W), jnp.float32),
        pltpu.VMEM((RPB, W), jnp.float32),
        pltpu.VMEM((RPB, W), jnp.float32),
        pltpu.VMEM((RPB, W), jnp.float32),
        pltpu.VMEM((RPB, W), jnp.float32),
        pltpu.VMEM((D * L,), jnp.float32),
        pltpu.VMEM((D * L,), jnp.float32),
        pltpu.VMEM((D * L,), jnp.float32),
        pltpu.VMEM((BPW,), jnp.float32),
        pltpu.VMEM((BPW,), jnp.float32),
        pltpu.VMEM((L,), jnp.float32),
        pltpu.SemaphoreType.DMA,
    ],
)(_sc_body)


CB = 8192


QB = CB // 4


def _tc_repack_body(in_ref, out_ref):
    y = jnp.transpose(in_ref[...])
    out_ref[...] = jnp.concatenate(
        [y[k * QB:(k + 1) * QB, :] for k in range(4)], axis=1)


def _tc_repack(ent_cm, n_ent):
    grid = (n_ent + CB - 1) // CB
    return pl.pallas_call(
        _tc_repack_body,
        grid=(grid,),
        in_specs=[pl.BlockSpec((D, CB), lambda j: (0, j))],
        out_specs=pl.BlockSpec((QB, W), lambda j: (j, 0)),
        out_shape=jax.ShapeDtypeStruct((grid * QB, W), jnp.float32),
    )(ent_cm)


def kernel(ent_embed, rel_embed, time_embed, pos_h, pos_r, pos_t,
           neg_h, neg_t, time_idx):
    ent_r = _tc_repack(ent_embed.T, ent_embed.shape[0])

    def prow(i):
        return (i // CB) * QB + i % QB

    def psub(i):
        return ((i % CB) // QB) * D

    rel_p = jnp.pad(rel_embed, ((0, 0), (0, W - D)))
    time_p = jnp.pad(time_embed, ((0, 0), (0, W - D)))
    idx_all = jnp.stack([time_idx, prow(pos_h), pos_r, prow(pos_t),
                         prow(neg_h), prow(neg_t)], axis=0)
    sub_all = jnp.stack([psub(pos_h), psub(pos_t),
                         psub(neg_h), psub(neg_t)], axis=0)
    scores, loss_parts = _sc_call(ent_r, rel_p, time_p, idx_all, sub_all)
    return scores, jnp.sum(loss_parts)


# speedup vs baseline: 1.4307x; 1.0323x over previous
import functools

import jax
import jax.numpy as jnp
from jax import lax
from jax.experimental import pallas as pl
from jax.experimental.pallas import tpu as pltpu
from jax.experimental.pallas import tpu_sc as plsc

D = 32
W = 128
B = 16384
NC, NS = 2, 16
NW = NC * NS
BPW = B // NW
RPB = 128
NCH = BPW // RPB
L = 16
NG = RPB // L
MARGIN = 1.0


def _sc_body(ent_hbm, rel_hbm, time_hbm, idx_hbm, sub_hbm,
             scores_out, loss_out,
             idx_v, sub_v, w_rows, h_rows, r_rows, t_rows, nh_rows, nt_rows,
             st, ut, wt, pos_buf, neg_buf, loss_buf, sem):
    wid = lax.axis_index("s") * NC + lax.axis_index("c")
    base = wid * BPW

    pltpu.sync_copy(idx_hbm.at[:, pl.ds(base, BPW)], idx_v)
    pltpu.sync_copy(sub_hbm.at[:, pl.ds(base, BPW)], sub_v)

    tables = (time_hbm, ent_hbm, rel_hbm, ent_hbm, ent_hbm, ent_hbm)
    dests = (w_rows, h_rows, r_rows, t_rows, nh_rows, nt_rows)
    zeros = jnp.zeros((L,), jnp.float32)
    lanes = lax.iota(jnp.int32, L)

    def chunk_body(c, lacc_c):
        cbase = c * RPB
        copies = []
        for a in range(6):
            copies.append(pltpu.async_copy(
                tables[a].at[idx_v.at[a, pl.ds(cbase, RPB)]],
                dests[a], sem))
        for cp in copies:
            cp.wait()

        def group_body(g, lacc):
            ridx = g * L + lanes
            gbase = cbase + g * L
            sh = sub_v[0, pl.ds(gbase, L)]
            stt = sub_v[1, pl.ds(gbase, L)]
            snh = sub_v[2, pl.ds(gbase, L)]
            snt = sub_v[3, pl.ds(gbase, L)]
            q = zeros
            dsum = zeros
            dusum = zeros
            for d in range(D):
                cidx = jnp.full((L,), d, jnp.int32)
                w_d = plsc.load_gather(w_rows, [ridx, cidx])
                r_d = plsc.load_gather(r_rows, [ridx, cidx])
                h_d = plsc.load_gather(h_rows, [ridx, sh + d])
                t_d = plsc.load_gather(t_rows, [ridx, stt + d])
                x_d = plsc.load_gather(nh_rows, [ridx, snh + d])
                y_d = plsc.load_gather(nt_rows, [ridx, snt + d])
                s_d = h_d + r_d - t_d
                u_d = x_d + r_d - y_d
                q = q + w_d * w_d
                dsum = dsum + w_d * s_d
                dusum = dusum + w_d * u_d
                wt[pl.ds(d * L, L)] = w_d
                st[pl.ds(d * L, L)] = s_d
                ut[pl.ds(d * L, L)] = u_d
            qi = 1.0 / (q + 1e-30)
            cs = dsum * qi
            cu = dusum * qi
            pos = zeros
            neg = zeros
            for d in range(D):
                w_d = wt[pl.ds(d * L, L)]
                s_d = st[pl.ds(d * L, L)]
                u_d = ut[pl.ds(d * L, L)]
                pos = pos + jnp.abs(s_d - cs * w_d)
                neg = neg + jnp.abs(u_d - cu * w_d)
            pos_buf[pl.ds(gbase, L)] = pos
            neg_buf[pl.ds(gbase, L)] = neg
            return lacc + jnp.maximum(pos + MARGIN - neg, 0.0)

        return lax.fori_loop(0, NG, group_body, lacc_c)

    lacc = lax.fori_loop(0, NCH, chunk_body, zeros)

    pltpu.sync_copy(pos_buf, scores_out.at[0, pl.ds(base, BPW)])
    pltpu.sync_copy(neg_buf, scores_out.at[1, pl.ds(base, BPW)])
    loss_buf[...] = lacc
    pltpu.sync_copy(loss_buf, loss_out.at[pl.ds(wid * L, L)])


_sc_call = functools.partial(
    pl.kernel,
    out_type=(
        jax.ShapeDtypeStruct((2, B), jnp.float32),
        jax.ShapeDtypeStruct((NW * L,), jnp.float32),
    ),
    mesh=plsc.VectorSubcoreMesh(core_axis_name="c", subcore_axis_name="s"),
    compiler_params=pltpu.CompilerParams(
        needs_layout_passes=False, use_tc_tiling_on_sc=False,
        disable_bounds_checks=True),
    scratch_types=[
        pltpu.VMEM((6, BPW), jnp.int32),
        pltpu.VMEM((4, BPW), jnp.int32),
        pltpu.VMEM((RPB, D), jnp.float32),
        pltpu.VMEM((RPB, W), jnp.float32),
        pltpu.VMEM((RPB, D), jnp.float32),
        pltpu.VMEM((RPB, W), jnp.float32),
        pltpu.VMEM((RPB, W), jnp.float32),
        pltpu.VMEM((RPB, W), jnp.float32),
        pltpu.VMEM((D * L,), jnp.float32),
        pltpu.VMEM((D * L,), jnp.float32),
        pltpu.VMEM((D * L,), jnp.float32),
        pltpu.VMEM((BPW,), jnp.float32),
        pltpu.VMEM((BPW,), jnp.float32),
        pltpu.VMEM((L,), jnp.float32),
        pltpu.SemaphoreType.DMA,
    ],
)(_sc_body)


CB = 32768


QB = CB // 4


def _tc_repack_body(in_ref, out_ref):
    y = jnp.transpose(in_ref[...])
    out_ref[...] = jnp.concatenate(
        [y[k * QB:(k + 1) * QB, :] for k in range(4)], axis=1)


def _tc_repack(ent_cm, n_ent):
    grid = (n_ent + CB - 1) // CB
    return pl.pallas_call(
        _tc_repack_body,
        grid=(grid,),
        in_specs=[pl.BlockSpec((D, CB), lambda j: (0, j))],
        out_specs=pl.BlockSpec((QB, W), lambda j: (j, 0)),
        out_shape=jax.ShapeDtypeStruct((grid * QB, W), jnp.float32),
    )(ent_cm)


def kernel(ent_embed, rel_embed, time_embed, pos_h, pos_r, pos_t,
           neg_h, neg_t, time_idx):
    ent_r = _tc_repack(ent_embed.T, ent_embed.shape[0])

    def prow(i):
        return (i // CB) * QB + i % QB

    def psub(i):
        return ((i % CB) // QB) * D

    idx_all = jnp.stack([time_idx, prow(pos_h), pos_r, prow(pos_t),
                         prow(neg_h), prow(neg_t)], axis=0)
    sub_all = jnp.stack([psub(pos_h), psub(pos_t),
                         psub(neg_h), psub(neg_t)], axis=0)
    scores, loss_parts = _sc_call(ent_r, rel_embed, time_embed, idx_all, sub_all)
    return scores, jnp.sum(loss_parts)


# speedup vs baseline: 1.4741x; 1.0304x over previous
import functools

import jax
import jax.numpy as jnp
from jax import lax
from jax.experimental import pallas as pl
from jax.experimental.pallas import tpu as pltpu
from jax.experimental.pallas import tpu_sc as plsc

D = 32
W = 128
B = 16384
NC, NS = 2, 16
NW = NC * NS
BPW = B // NW
RPB = 64
NCH = BPW // RPB
L = 16
NG = RPB // L
MARGIN = 1.0


def _sc_body(ent_hbm, rel_hbm, time_hbm, idx_hbm, sub_hbm,
             scores_out, loss_out,
             idx_v, sub_v,
             w_a, h_a, r_a, t_a, nh_a, nt_a,
             w_b, h_b, r_b, t_b, nh_b, nt_b,
             st, ut, wt, pos_buf, neg_buf, loss_buf, sem_a, sem_b):
    wid = lax.axis_index("s") * NC + lax.axis_index("c")
    base = wid * BPW

    pltpu.sync_copy(idx_hbm.at[:, pl.ds(base, BPW)], idx_v)
    pltpu.sync_copy(sub_hbm.at[:, pl.ds(base, BPW)], sub_v)

    tables = (time_hbm, ent_hbm, rel_hbm, ent_hbm, ent_hbm, ent_hbm)
    bufs_a = (w_a, h_a, r_a, t_a, nh_a, nt_a)
    bufs_b = (w_b, h_b, r_b, t_b, nh_b, nt_b)
    zeros = jnp.zeros((L,), jnp.float32)
    lanes = lax.iota(jnp.int32, L)

    def _copies(c, bufs, sem):
        return [pltpu.make_async_copy(
            tables[a].at[idx_v.at[a, pl.ds(c * RPB, RPB)]], bufs[a], sem)
            for a in range(6)]

    def issue(c, bufs, sem):
        for cp in _copies(c, bufs, sem):
            cp.start()

    def drain(c, bufs, sem):
        for cp in _copies(c, bufs, sem):
            cp.wait()

    def compute(c, bufs, lacc_c):
        cbase = c * RPB
        (w_rows, h_rows, r_rows, t_rows, nh_rows, nt_rows) = bufs

        def group_body(g, lacc):
            ridx = g * L + lanes
            gbase = cbase + g * L
            sh = sub_v[0, pl.ds(gbase, L)]
            stt = sub_v[1, pl.ds(gbase, L)]
            snh = sub_v[2, pl.ds(gbase, L)]
            snt = sub_v[3, pl.ds(gbase, L)]
            q = zeros
            dsum = zeros
            dusum = zeros
            for d in range(D):
                cidx = jnp.full((L,), d, jnp.int32)
                w_d = plsc.load_gather(w_rows, [ridx, cidx])
                r_d = plsc.load_gather(r_rows, [ridx, cidx])
                h_d = plsc.load_gather(h_rows, [ridx, sh + d])
                t_d = plsc.load_gather(t_rows, [ridx, stt + d])
                x_d = plsc.load_gather(nh_rows, [ridx, snh + d])
                y_d = plsc.load_gather(nt_rows, [ridx, snt + d])
                s_d = h_d + r_d - t_d
                u_d = x_d + r_d - y_d
                q = q + w_d * w_d
                dsum = dsum + w_d * s_d
                dusum = dusum + w_d * u_d
                wt[pl.ds(d * L, L)] = w_d
                st[pl.ds(d * L, L)] = s_d
                ut[pl.ds(d * L, L)] = u_d
            qi = 1.0 / (q + 1e-30)
            cs = dsum * qi
            cu = dusum * qi
            pos = zeros
            neg = zeros
            for d in range(D):
                w_d = wt[pl.ds(d * L, L)]
                s_d = st[pl.ds(d * L, L)]
                u_d = ut[pl.ds(d * L, L)]
                pos = pos + jnp.abs(s_d - cs * w_d)
                neg = neg + jnp.abs(u_d - cu * w_d)
            pos_buf[pl.ds(gbase, L)] = pos
            neg_buf[pl.ds(gbase, L)] = neg
            return lacc + jnp.maximum(pos + MARGIN - neg, 0.0)

        return lax.fori_loop(0, NG, group_body, lacc_c)

    issue(0, bufs_a, sem_a)

    def pair_body(t, lacc):
        c0 = 2 * t
        issue(c0 + 1, bufs_b, sem_b)
        drain(c0, bufs_a, sem_a)
        lacc = compute(c0, bufs_a, lacc)
        lax.cond(t < NCH // 2 - 1,
                 lambda: issue(c0 + 2, bufs_a, sem_a), lambda: None)
        drain(c0 + 1, bufs_b, sem_b)
        return compute(c0 + 1, bufs_b, lacc)

    lacc = lax.fori_loop(0, NCH // 2, pair_body, zeros)

    pltpu.sync_copy(pos_buf, scores_out.at[0, pl.ds(base, BPW)])
    pltpu.sync_copy(neg_buf, scores_out.at[1, pl.ds(base, BPW)])
    loss_buf[...] = lacc
    pltpu.sync_copy(loss_buf, loss_out.at[pl.ds(wid * L, L)])


_sc_call = functools.partial(
    pl.kernel,
    out_type=(
        jax.ShapeDtypeStruct((2, B), jnp.float32),
        jax.ShapeDtypeStruct((NW * L,), jnp.float32),
    ),
    mesh=plsc.VectorSubcoreMesh(core_axis_name="c", subcore_axis_name="s"),
    compiler_params=pltpu.CompilerParams(
        needs_layout_passes=False, use_tc_tiling_on_sc=False,
        disable_bounds_checks=True),
    scratch_types=[
        pltpu.VMEM((6, BPW), jnp.int32),
        pltpu.VMEM((4, BPW), jnp.int32),
        pltpu.VMEM((RPB, D), jnp.float32),
        pltpu.VMEM((RPB, W), jnp.float32),
        pltpu.VMEM((RPB, D), jnp.float32),
        pltpu.VMEM((RPB, W), jnp.float32),
        pltpu.VMEM((RPB, W), jnp.float32),
        pltpu.VMEM((RPB, W), jnp.float32),
        pltpu.VMEM((RPB, D), jnp.float32),
        pltpu.VMEM((RPB, W), jnp.float32),
        pltpu.VMEM((RPB, D), jnp.float32),
        pltpu.VMEM((RPB, W), jnp.float32),
        pltpu.VMEM((RPB, W), jnp.float32),
        pltpu.VMEM((RPB, W), jnp.float32),
        pltpu.VMEM((D * L,), jnp.float32),
        pltpu.VMEM((D * L,), jnp.float32),
        pltpu.VMEM((D * L,), jnp.float32),
        pltpu.VMEM((BPW,), jnp.float32),
        pltpu.VMEM((BPW,), jnp.float32),
        pltpu.VMEM((L,), jnp.float32),
        pltpu.SemaphoreType.DMA,
        pltpu.SemaphoreType.DMA,
    ],
)(_sc_body)


CB = 32768


QB = CB // 4


def _tc_repack_body(in_ref, out_ref):
    y = jnp.transpose(in_ref[...])
    out_ref[...] = jnp.concatenate(
        [y[k * QB:(k + 1) * QB, :] for k in range(4)], axis=1)


def _tc_repack(ent_cm, n_ent):
    grid = (n_ent + CB - 1) // CB
    return pl.pallas_call(
        _tc_repack_body,
        grid=(grid,),
        in_specs=[pl.BlockSpec((D, CB), lambda j: (0, j))],
        out_specs=pl.BlockSpec((QB, W), lambda j: (j, 0)),
        out_shape=jax.ShapeDtypeStruct((grid * QB, W), jnp.float32),
    )(ent_cm)


def kernel(ent_embed, rel_embed, time_embed, pos_h, pos_r, pos_t,
           neg_h, neg_t, time_idx):
    ent_r = _tc_repack(ent_embed.T, ent_embed.shape[0])

    def prow(i):
        return (i // CB) * QB + i % QB

    def psub(i):
        return ((i % CB) // QB) * D

    idx_all = jnp.stack([time_idx, prow(pos_h), pos_r, prow(pos_t),
                         prow(neg_h), prow(neg_t)], axis=0)
    sub_all = jnp.stack([psub(pos_h), psub(pos_t),
                         psub(neg_h), psub(neg_t)], axis=0)
    scores, loss_parts = _sc_call(ent_r, rel_embed, time_embed, idx_all, sub_all)
    return scores, jnp.sum(loss_parts)


# speedup vs baseline: 1.7202x; 1.1669x over previous
import functools

import jax
import jax.numpy as jnp
from jax import lax
from jax.experimental import pallas as pl
from jax.experimental.pallas import tpu as pltpu
from jax.experimental.pallas import tpu_sc as plsc

D = 32
W = 128
B = 16384
NC, NS = 2, 16
NW = NC * NS
BPW = B // NW
RPB = 64
NCH = BPW // RPB
L = 16
NG = RPB // L
MARGIN = 1.0


def _sc_body(ent_hbm, rel_hbm, time_hbm, idx_hbm, sub_hbm,
             scores_out, loss_out,
             idx_v, sub_v,
             w_a, h_a, r_a, t_a, nh_a, nt_a,
             w_b, h_b, r_b, t_b, nh_b, nt_b,
             st, ut, wt, pos_buf, neg_buf, loss_buf, sem_a, sem_b):
    wid = lax.axis_index("s") * NC + lax.axis_index("c")
    base = wid * BPW

    pltpu.sync_copy(idx_hbm.at[:, pl.ds(base, BPW)], idx_v)
    pltpu.sync_copy(sub_hbm.at[:, pl.ds(base, BPW)], sub_v)

    tables = (time_hbm, ent_hbm, rel_hbm, ent_hbm, ent_hbm, ent_hbm)
    bufs_a = (w_a, h_a, r_a, t_a, nh_a, nt_a)
    bufs_b = (w_b, h_b, r_b, t_b, nh_b, nt_b)
    zeros = jnp.zeros((L,), jnp.float32)
    lanes = lax.iota(jnp.int32, L)

    def _copies(c, bufs, sem):
        return [pltpu.make_async_copy(
            tables[a].at[idx_v.at[a, pl.ds(c * RPB, RPB)]], bufs[a], sem)
            for a in range(6)]

    def issue(c, bufs, sem):
        for cp in _copies(c, bufs, sem):
            cp.start()

    def drain(c, bufs, sem):
        for cp in _copies(c, bufs, sem):
            cp.wait()

    def compute(c, bufs, lacc_c):
        cbase = c * RPB
        (w_rows, h_rows, r_rows, t_rows, nh_rows, nt_rows) = bufs
        lo = pl.ds(0, L)
        hi = pl.ds(L, L)

        def group_body(g, lacc):
            gbase = cbase + g * L
            shv = sub_v[0, pl.ds(gbase, L)]
            sttv = sub_v[1, pl.ds(gbase, L)]
            snhv = sub_v[2, pl.ds(gbase, L)]
            sntv = sub_v[3, pl.ds(gbase, L)]
            pos_v = zeros
            neg_v = zeros
            for j in range(L):
                li = g * L + j
                sh = shv[j]
                stt = sttv[j]
                snh = snhv[j]
                snt = sntv[j]
                wa, wb = w_rows[li, lo], w_rows[li, hi]
                ra, rb = r_rows[li, lo], r_rows[li, hi]
                ha = h_rows[li, pl.ds(sh, L)]
                hb = h_rows[li, pl.ds(sh + L, L)]
                ta = t_rows[li, pl.ds(stt, L)]
                tb = t_rows[li, pl.ds(stt + L, L)]
                xa = nh_rows[li, pl.ds(snh, L)]
                xb = nh_rows[li, pl.ds(snh + L, L)]
                ya = nt_rows[li, pl.ds(snt, L)]
                yb = nt_rows[li, pl.ds(snt + L, L)]
                sa = ha + ra - ta
                sb = hb + rb - tb
                ua = xa + ra - ya
                ub = xb + rb - yb
                q = jnp.sum(wa * wa + wb * wb)
                dsum = jnp.sum(wa * sa + wb * sb)
                dusum = jnp.sum(wa * ua + wb * ub)
                qi = 1.0 / (q + zeros + 1e-30)
                cs = dsum * qi
                cu = dusum * qi
                pos = jnp.sum(jnp.abs(sa - cs * wa) + jnp.abs(sb - cs * wb))
                neg = jnp.sum(jnp.abs(ua - cu * wa) + jnp.abs(ub - cu * wb))
                mask = lanes == j
                pos_v = jnp.where(mask, pos, pos_v)
                neg_v = jnp.where(mask, neg, neg_v)
            pos_buf[pl.ds(gbase, L)] = pos_v
            neg_buf[pl.ds(gbase, L)] = neg_v
            return lacc + jnp.maximum(pos_v + MARGIN - neg_v, 0.0)

        return lax.fori_loop(0, NG, group_body, lacc_c)

    issue(0, bufs_a, sem_a)

    def pair_body(t, lacc):
        c0 = 2 * t
        issue(c0 + 1, bufs_b, sem_b)
        drain(c0, bufs_a, sem_a)
        lacc = compute(c0, bufs_a, lacc)
        lax.cond(t < NCH // 2 - 1,
                 lambda: issue(c0 + 2, bufs_a, sem_a), lambda: None)
        drain(c0 + 1, bufs_b, sem_b)
        return compute(c0 + 1, bufs_b, lacc)

    lacc = lax.fori_loop(0, NCH // 2, pair_body, zeros)

    pltpu.sync_copy(pos_buf, scores_out.at[0, pl.ds(base, BPW)])
    pltpu.sync_copy(neg_buf, scores_out.at[1, pl.ds(base, BPW)])
    loss_buf[...] = lacc
    pltpu.sync_copy(loss_buf, loss_out.at[pl.ds(wid * L, L)])


_sc_call = functools.partial(
    pl.kernel,
    out_type=(
        jax.ShapeDtypeStruct((2, B), jnp.float32),
        jax.ShapeDtypeStruct((NW * L,), jnp.float32),
    ),
    mesh=plsc.VectorSubcoreMesh(core_axis_name="c", subcore_axis_name="s"),
    compiler_params=pltpu.CompilerParams(
        needs_layout_passes=False, use_tc_tiling_on_sc=False,
        disable_bounds_checks=True),
    scratch_types=[
        pltpu.VMEM((6, BPW), jnp.int32),
        pltpu.VMEM((4, BPW), jnp.int32),
        pltpu.VMEM((RPB, D), jnp.float32),
        pltpu.VMEM((RPB, W), jnp.float32),
        pltpu.VMEM((RPB, D), jnp.float32),
        pltpu.VMEM((RPB, W), jnp.float32),
        pltpu.VMEM((RPB, W), jnp.float32),
        pltpu.VMEM((RPB, W), jnp.float32),
        pltpu.VMEM((RPB, D), jnp.float32),
        pltpu.VMEM((RPB, W), jnp.float32),
        pltpu.VMEM((RPB, D), jnp.float32),
        pltpu.VMEM((RPB, W), jnp.float32),
        pltpu.VMEM((RPB, W), jnp.float32),
        pltpu.VMEM((RPB, W), jnp.float32),
        pltpu.VMEM((D * L,), jnp.float32),
        pltpu.VMEM((D * L,), jnp.float32),
        pltpu.VMEM((D * L,), jnp.float32),
        pltpu.VMEM((BPW,), jnp.float32),
        pltpu.VMEM((BPW,), jnp.float32),
        pltpu.VMEM((L,), jnp.float32),
        pltpu.SemaphoreType.DMA,
        pltpu.SemaphoreType.DMA,
    ],
)(_sc_body)


CB = 32768


QB = CB // 4


def _tc_repack_body(in_ref, out_ref):
    y = jnp.transpose(in_ref[...])
    out_ref[...] = jnp.concatenate(
        [y[k * QB:(k + 1) * QB, :] for k in range(4)], axis=1)


def _tc_repack(ent_cm, n_ent):
    grid = (n_ent + CB - 1) // CB
    return pl.pallas_call(
        _tc_repack_body,
        grid=(grid,),
        in_specs=[pl.BlockSpec((D, CB), lambda j: (0, j))],
        out_specs=pl.BlockSpec((QB, W), lambda j: (j, 0)),
        out_shape=jax.ShapeDtypeStruct((grid * QB, W), jnp.float32),
    )(ent_cm)


def kernel(ent_embed, rel_embed, time_embed, pos_h, pos_r, pos_t,
           neg_h, neg_t, time_idx):
    ent_r = _tc_repack(ent_embed.T, ent_embed.shape[0])

    def prow(i):
        return (i // CB) * QB + i % QB

    def psub(i):
        return ((i % CB) // QB) * D

    idx_all = jnp.stack([time_idx, prow(pos_h), pos_r, prow(pos_t),
                         prow(neg_h), prow(neg_t)], axis=0)
    sub_all = jnp.stack([psub(pos_h), psub(pos_t),
                         psub(neg_h), psub(neg_t)], axis=0)
    scores, loss_parts = _sc_call(ent_r, rel_embed, time_embed, idx_all, sub_all)
    return scores, jnp.sum(loss_parts)
